# Initial kernel scaffold; baseline (speedup 1.0000x reference)
import jax
import jax.numpy as jnp
from jax.experimental import pallas as pl


def kernel(x, edge_index, edge_attr, mlp1_w0, mlp1_b0, mlp1_w1, mlp1_b1, mlp1_w2, mlp1_b2, mlp1_w3, mlp1_b3, root1, conv1_b, mlp2_w0, mlp2_b0, mlp2_w1, mlp2_b1, mlp2_w2, mlp2_b2, mlp2_w3, mlp2_b3, root2, conv2_b, fc1_w, fc1_b, fc2_w, fc2_b):
    raise NotImplementedError("write your pallas kernel here")


# speedup vs baseline: 2.2317x; 2.2317x over previous
import jax
import jax.numpy as jnp
from jax import lax
from jax.experimental import pallas as pl
from jax.experimental.pallas import tpu as pltpu
from jax.experimental.pallas import tpu_sc as plsc

_NC = 2
_NS = 16
_NW = _NC * _NS
_CHUNK = 128
_EB = 512
_NB = 512


def _sc_mesh():
    return plsc.VectorSubcoreMesh(core_axis_name="c", subcore_axis_name="s",
                                  num_cores=_NC, num_subcores=_NS)


def _sc_gather(table, idx3):
    _, F = table.shape
    _, nch, _ = idx3.shape
    per = nch * _CHUNK

    def body(table_ref, idx_ref, out_ref, idx_v, rows_v, sem):
        wid = lax.axis_index("c") * _NS + lax.axis_index("s")
        pltpu.sync_copy(idx_ref.at[wid], idx_v)
        base = wid * per
        for j in range(nch):
            pltpu.async_copy(table_ref.at[idx_v.at[j]], rows_v, sem).wait()
            pltpu.sync_copy(rows_v, out_ref.at[pl.ds(base + j * _CHUNK, _CHUNK)])

    return pl.kernel(
        body,
        out_type=jax.ShapeDtypeStruct((_NW * per, F), jnp.float32),
        mesh=_sc_mesh(),
        scratch_types=[
            pltpu.VMEM((nch, _CHUNK), jnp.int32),
            pltpu.VMEM((_CHUNK, F), jnp.float32),
            pltpu.SemaphoreType.DMA,
        ],
        name="sc_gather",
    )(table, idx3)


def _sc_scatter(msg, idx3, npad):
    _, W = msg.shape
    _, nch, _ = idx3.shape
    per = nch * _CHUNK
    rpt = npad // _NS

    def body(msg_ref, idx_ref, zeros_ref, out_ref, idx_v, dbuf, acc, sem):
        c = lax.axis_index("c")
        s = lax.axis_index("s")
        wid = c * _NS + s
        r0 = s * rpt
        pltpu.sync_copy(zeros_ref.at[pl.ds(r0, rpt)], acc.at[pl.ds(r0, rpt)])
        pltpu.sync_copy(idx_ref.at[wid], idx_v)
        plsc.subcore_barrier()
        base = wid * per
        for j in range(nch):
            pltpu.sync_copy(msg_ref.at[pl.ds(base + j * _CHUNK, _CHUNK)], dbuf)
            pltpu.sync_copy(dbuf, acc.at[idx_v.at[j]], add=True)
        plsc.subcore_barrier()
        pltpu.sync_copy(acc.at[pl.ds(r0, rpt)], out_ref.at[c, pl.ds(r0, rpt)])

    zeros = jnp.zeros((npad, W), jnp.float32)
    return pl.kernel(
        body,
        out_type=jax.ShapeDtypeStruct((_NC, npad, W), jnp.float32),
        mesh=_sc_mesh(),
        scratch_types=[
            pltpu.VMEM((nch, _CHUNK), jnp.int32),
            pltpu.VMEM((_CHUNK, W), jnp.float32),
            pltpu.VMEM_SHARED((npad, W), jnp.float32),
            pltpu.SemaphoreType.DMA,
        ],
        name="sc_scatter",
    )(msg, idx3, zeros)


def _full(shape):
    return pl.BlockSpec(shape, lambda i: (0,) * len(shape))


def _tc_edge(ea, xs, IN, ws, bs):
    EP, _ = ea.shape
    hid = ws[3].shape[0]
    H = ws[3].shape[1] // IN
    W = 128

    def body(ea_ref, xs_ref, w0r, b0r, w1r, b1r, w2r, b2r, w3r, b3r, Rr,
             out_ref):
        f32, bf = jnp.float32, jnp.bfloat16
        h = jnp.maximum(jnp.dot(ea_ref[...].astype(bf), w0r[...],
                                preferred_element_type=f32) + b0r[...], 0.0)
        h = jnp.maximum(jnp.dot(h.astype(bf), w1r[...],
                                preferred_element_type=f32) + b1r[...], 0.0)
        h = jnp.maximum(jnp.dot(h.astype(bf), w2r[...],
                                preferred_element_type=f32) + b2r[...], 0.0)
        P = jnp.dot(h.astype(bf), w3r[...], preferred_element_type=f32)
        xsb = xs_ref[...][:, :IN].astype(bf)
        xrep = jnp.dot(xsb, Rr[...], preferred_element_type=f32)
        s = xrep[:, 0:128] * P[:, 0:128]
        for j in range(1, (IN * H) // 128):
            s = s + xrep[:, 128 * j:128 * (j + 1)] * P[:, 128 * j:128 * (j + 1)]
        msg = s[:, :H] + s[:, H:] + jnp.dot(xsb, b3r[...],
                                            preferred_element_type=f32)
        out_ref[...] = jnp.concatenate(
            [msg, jnp.ones((_EB, W - H), f32)], axis=1)

    ii = jax.lax.broadcasted_iota(jnp.int32, (IN, IN * H), 0)
    jj = jax.lax.broadcasted_iota(jnp.int32, (IN, IN * H), 1)
    R = (jj // H == ii).astype(jnp.bfloat16)

    f16 = ea.shape[1]
    return pl.pallas_call(
        body,
        grid=(EP // _EB,),
        in_specs=[
            pl.BlockSpec((_EB, f16), lambda i: (i, 0)),
            pl.BlockSpec((_EB, 128), lambda i: (i, 0)),
            _full(ws[0].shape), _full((1, hid)),
            _full(ws[1].shape), _full((1, hid)),
            _full(ws[2].shape), _full((1, hid)),
            _full(ws[3].shape), _full((IN, H)),
            _full(R.shape),
        ],
        out_specs=pl.BlockSpec((_EB, W), lambda i: (i, 0)),
        out_shape=jax.ShapeDtypeStruct((EP, W), jnp.float32),
    )(ea, xs, ws[0].astype(jnp.bfloat16), bs[0].reshape(1, -1),
      ws[1].astype(jnp.bfloat16), bs[1].reshape(1, -1),
      ws[2].astype(jnp.bfloat16), bs[2].reshape(1, -1),
      ws[3].astype(jnp.bfloat16), bs[3].reshape(IN, H).astype(jnp.bfloat16),
      R)


def _tc_node1(p, x, IN, root, bias):
    npad = x.shape[0]
    H = root.shape[1]
    W = p.shape[2]

    def body(p0r, p1r, xr, rootr, br, h_ref, cnt_ref):
        s = p0r[0] + p1r[0]
        cntc = jnp.maximum(s[:, H:H + 1], 1.0)
        z = s[:, :H] / cntc
        z = z + jnp.dot(xr[...][:, :IN], rootr[...],
                        preferred_element_type=jnp.float32)
        z = z + br[...]
        h = jnp.where(z > 0, z, jnp.exp(z) - 1.0)
        h_ref[...] = jnp.concatenate(
            [h, jnp.zeros((_NB, 128 - H), jnp.float32)], axis=1)
        cnt_ref[...] = cntc

    return pl.pallas_call(
        body,
        grid=(npad // _NB,),
        in_specs=[
            pl.BlockSpec((1, _NB, W), lambda i: (0, i, 0)),
            pl.BlockSpec((1, _NB, W), lambda i: (1, i, 0)),
            pl.BlockSpec((_NB, 128), lambda i: (i, 0)),
            _full(root.shape), _full((1, H)),
        ],
        out_specs=[pl.BlockSpec((_NB, 128), lambda i: (i, 0)),
                   pl.BlockSpec((_NB, 1), lambda i: (i, 0))],
        out_shape=[jax.ShapeDtypeStruct((npad, 128), jnp.float32),
                   jax.ShapeDtypeStruct((npad, 1), jnp.float32)],
    )(p, p, x, root, bias.reshape(1, -1))


def _tc_node2(p, cntc, hprev, root, bias):
    npad = hprev.shape[0]
    H = root.shape[1]
    W = p.shape[2]

    def body(p0r, p1r, cr, hr, rootr, br, out_ref):
        s = p0r[0] + p1r[0]
        z = s[:, :H] / cr[...]
        z = z + jnp.dot(hr[...][:, :H], rootr[...],
                        preferred_element_type=jnp.float32)
        z = z + br[...]
        out_ref[...] = jnp.where(z > 0, z, jnp.exp(z) - 1.0)

    return pl.pallas_call(
        body,
        grid=(npad // _NB,),
        in_specs=[
            pl.BlockSpec((1, _NB, W), lambda i: (0, i, 0)),
            pl.BlockSpec((1, _NB, W), lambda i: (1, i, 0)),
            pl.BlockSpec((_NB, 1), lambda i: (i, 0)),
            pl.BlockSpec((_NB, 128), lambda i: (i, 0)),
            _full(root.shape), _full((1, H)),
        ],
        out_specs=pl.BlockSpec((_NB, H), lambda i: (i, 0)),
        out_shape=jax.ShapeDtypeStruct((npad, H), jnp.float32),
    )(p, p, cntc, hprev, root, bias.reshape(1, -1))


def _tc_head(h, fc1_w, fc1_b, fc2_w, fc2_b):
    npad, H = h.shape
    NCLS = fc2_w.shape[1]

    def body(hr, w1r, b1r, w2r, b2r, out_ref):
        f32 = jnp.float32
        z = jnp.dot(hr[...], w1r[...], preferred_element_type=f32) + b1r[...]
        t = jnp.where(z > 0, z, jnp.exp(z) - 1.0)
        logits = jnp.dot(t, w2r[...], preferred_element_type=f32) + b2r[...]
        m = jnp.max(logits, axis=1, keepdims=True)
        e = jnp.exp(logits - m)
        lse = jnp.log(jnp.sum(e, axis=1, keepdims=True)) + m
        out_ref[...] = logits - lse

    return pl.pallas_call(
        body,
        grid=(npad // _NB,),
        in_specs=[
            pl.BlockSpec((_NB, H), lambda i: (i, 0)),
            _full(fc1_w.shape), _full((1, H)),
            _full(fc2_w.shape), _full((1, NCLS)),
        ],
        out_specs=pl.BlockSpec((_NB, NCLS), lambda i: (i, 0)),
        out_shape=jax.ShapeDtypeStruct((npad, NCLS), jnp.float32),
    )(h, fc1_w, fc1_b.reshape(1, -1), fc2_w, fc2_b.reshape(1, -1))


def kernel(x, edge_index, edge_attr,
           mlp1_w0, mlp1_b0, mlp1_w1, mlp1_b1, mlp1_w2, mlp1_b2,
           mlp1_w3, mlp1_b3, root1, conv1_b,
           mlp2_w0, mlp2_b0, mlp2_w1, mlp2_b1, mlp2_w2, mlp2_b2,
           mlp2_w3, mlp2_b3, root2, conv2_b,
           fc1_w, fc1_b, fc2_w, fc2_b):
    N, _ = x.shape
    E = edge_index.shape[1]
    H = root1.shape[1]

    nch = -(-E // (_NW * _CHUNK))
    epad = _NW * nch * _CHUNK
    npad = -(-N // _NB) * _NB
    pad_e = epad - E

    IN1 = x.shape[1]

    src = edge_index[0]
    dst = edge_index[1]
    src3 = jnp.concatenate(
        [src, jnp.zeros((pad_e,), jnp.int32)]).reshape(_NW, nch, _CHUNK)
    dst3 = jnp.concatenate(
        [dst, jnp.full((pad_e,), N, jnp.int32)]).reshape(_NW, nch, _CHUNK)
    ea_p = jnp.pad(edge_attr, ((0, pad_e), (0, 0)))
    x_p = jnp.pad(x, ((0, npad - N), (0, 128 - IN1)))

    xsrc = _sc_gather(x_p, src3)
    msg1 = _tc_edge(ea_p, xsrc, IN1,
                    (mlp1_w0, mlp1_w1, mlp1_w2, mlp1_w3),
                    (mlp1_b0, mlp1_b1, mlp1_b2, mlp1_b3))
    part1 = _sc_scatter(msg1, dst3, npad)
    h, cntc = _tc_node1(part1, x_p, IN1, root1, conv1_b)

    hsrc = _sc_gather(h, src3)
    msg2 = _tc_edge(ea_p, hsrc, H,
                    (mlp2_w0, mlp2_w1, mlp2_w2, mlp2_w3),
                    (mlp2_b0, mlp2_b1, mlp2_b2, mlp2_b3))
    part2 = _sc_scatter(msg2, dst3, npad)
    h2 = _tc_node2(part2, cntc, h, root2, conv2_b)

    out = _tc_head(h2, fc1_w, fc1_b, fc2_w, fc2_b)
    return out[:N]
